# baseline (speedup 1.0000x reference)
import functools

import jax
import jax.numpy as jnp
from jax import lax
from jax.experimental import pallas as pl
from jax.experimental.pallas import tpu as pltpu
from jax.experimental.pallas import tpu_sc as plsc

B, N, D_PTS, D, K = 4, 2048, 64, 128, 24
F32 = jnp.float32


def _mlinear_body(feats, WcT, bc, W1, W2):
    mf = jnp.mean(feats, axis=0, keepdims=True)
    mf_rows = jnp.broadcast_to(mf, (D, D))
    mf_cols = mf_rows.T
    w = mf_rows * W1 - mf_cols * W2
    w = jnp.dot(w, WcT, preferred_element_type=F32) + bc
    denom = jnp.sum(jnp.abs(w) + 1e-05, axis=-1, keepdims=True)
    w = w / denom
    return jnp.dot(feats, w, preferred_element_type=F32)


def _k1_body(feats_ref, xyz_ref, fc1_wT_ref, fc1_b_ref,
             WcT_ref, bc_ref, W1_ref, W2_ref, fdw1T_ref,
             x_ref, q_ref, p_ref, tbl_ref):
    x = jnp.dot(feats_ref[0], fc1_wT_ref[...],
                preferred_element_type=F32) + fc1_b_ref[...]
    x_ref[0] = x
    q_ref[0] = _mlinear_body(x, WcT_ref[0], bc_ref[0], W1_ref[0], W2_ref[0])
    kf = _mlinear_body(x, WcT_ref[1], bc_ref[1], W1_ref[1], W2_ref[1])
    vf = _mlinear_body(x, WcT_ref[2], bc_ref[2], W1_ref[2], W2_ref[2])
    p = jnp.dot(xyz_ref[0], fdw1T_ref[...], preferred_element_type=F32)
    p_ref[0] = p
    def hi16(a):
        return jnp.bitwise_and(
            lax.bitcast_convert_type(
                a.astype(jnp.bfloat16).astype(F32), jnp.int32),
            jnp.int32(-65536))
    kv = jnp.bitwise_or(hi16(kf), lax.shift_right_logical(hi16(vf), 16))
    tbl_ref[0] = jnp.concatenate([kv, hi16(p)], axis=-1)


def _k5_body(res_ref, x_ref, WcT_ref, bc_ref, W1_ref, W2_ref, out_ref):
    out_ref[0] = _mlinear_body(res_ref[0], WcT_ref[...], bc_ref[...],
                               W1_ref[...], W2_ref[...]) + x_ref[0]


NB_TOPK = 1024


def _k2_body(xyz_ref, xyzT_ref, idx_ref):
    b = pl.program_id(0)
    xyz_blk = xyz_ref[0]
    xyzT = xyzT_ref[0]
    dot = jnp.dot(xyz_blk, xyzT, preferred_element_type=F32)
    sq_blk = jnp.sum(xyz_blk * xyz_blk, axis=1, keepdims=True)
    sq_row = jnp.sum(xyzT * xyzT, axis=0, keepdims=True)
    d = (sq_blk + sq_row) - 2.0 * dot
    INF = jnp.float32(jnp.inf)
    NF = jnp.float32(N)
    iota = (lax.broadcasted_iota(jnp.int32, (NB_TOPK, N), 1)).astype(F32)
    cols = []
    for _ in range(K):
        m = jnp.min(d, axis=1, keepdims=True)
        j = jnp.min(jnp.where(d == m, iota, NF), axis=1, keepdims=True)
        cols.append(j)
        d = jnp.where(iota == j, INF, d)
    idx = jnp.concatenate(cols, axis=1).astype(jnp.int32)
    idx_ref[0] = idx + b * N


TW = 2 * D
NW = 32
CHUNK = 128


def _sc_gather(table, idx_flat, rows_total):
    rows_per_w = rows_total // NW
    n_chunks = rows_per_w // CHUNK
    mesh = plsc.VectorSubcoreMesh(core_axis_name="c", subcore_axis_name="s")

    @functools.partial(
        pl.kernel,
        mesh=mesh,
        out_type=jax.ShapeDtypeStruct((rows_total, TW), jnp.int32),
        scratch_types=[
            pltpu.VMEM((rows_per_w,), jnp.int32),
            pltpu.VMEM((2, CHUNK, TW), jnp.int32),
            pltpu.SemaphoreType.DMA,
            pltpu.SemaphoreType.DMA,
        ],
    )
    def gather_kernel(table_hbm, idx_hbm, out_hbm, idx_v, rows_v, s0, s1):
        wid = lax.axis_index("s") * 2 + lax.axis_index("c")
        w_base = wid * rows_per_w
        pltpu.sync_copy(idx_hbm.at[pl.ds(w_base, rows_per_w)], idx_v)
        sems = (s0, s1)

        def start(c, buf):
            pltpu.async_copy(
                table_hbm.at[idx_v.at[pl.ds(c * CHUNK, CHUNK)]],
                rows_v.at[buf], sems[buf])

        def finish(c, buf):
            pltpu.make_async_copy(
                table_hbm.at[idx_v.at[pl.ds(c * CHUNK, CHUNK)]],
                rows_v.at[buf], sems[buf]).wait()
            pltpu.sync_copy(rows_v.at[buf],
                            out_hbm.at[pl.ds(w_base + c * CHUNK, CHUNK)])

        start(0, 0)

        def body(i, carry):
            c = i * 2
            start(c + 1, 1)
            finish(c, 0)

            @pl.when(c + 2 < n_chunks)
            def _():
                start(c + 2, 0)

            finish(c + 1, 1)
            return carry

        lax.fori_loop(0, n_chunks // 2, body, 0)

    return gather_kernel(table, idx_flat)


NB_ATT = 128


def _k4_body(q_ref, g_ref, p_ref,
             fdb1_ref, fdw2T_ref, fdb2_ref,
             fgw1T_ref, fgb1_ref, fgw2T_ref, fgb2_ref,
             attn_ref, res_ref):
    rows = NB_ATT * K
    g = g_ref[0]
    kvg = g[:, :D]
    kg = lax.bitcast_convert_type(
        jnp.bitwise_and(kvg, jnp.int32(-65536)), F32)
    vg = lax.bitcast_convert_type(lax.shift_left(kvg, 16), F32)
    pg = lax.bitcast_convert_type(g[:, D:], F32)

    p_blk = p_ref[0]
    p_rep = jnp.reshape(
        jnp.broadcast_to(p_blk[:, None, :], (NB_ATT, K, D)), (rows, D))
    h1 = jax.nn.relu(p_rep - pg + fdb1_ref[...])
    pos = jnp.dot(h1, fdw2T_ref[...],
                  preferred_element_type=F32) + fdb2_ref[...]

    q_rep = jnp.reshape(
        jnp.broadcast_to(q_ref[0][:, None, :], (NB_ATT, K, D)), (rows, D))
    pre = q_rep - kg + pos
    h2 = jax.nn.relu(
        jnp.dot(pre, fgw1T_ref[...], preferred_element_type=F32)
        + fgb1_ref[...])
    attn = jnp.dot(h2, fgw2T_ref[...],
                   preferred_element_type=F32) + fgb2_ref[...]
    attn = attn / jnp.sum(jnp.abs(attn) + 1e-05, axis=-1, keepdims=True)
    attn_ref[0] = attn
    wv = attn * (vg + pos)
    res_ref[0] = jnp.sum(jnp.reshape(wv, (NB_ATT, K, D)), axis=1)


def kernel(features, xyz, fc1_w, fc1_b,
           qs_Wc, qs_bc, qs_W1, qs_W2,
           kk_Wc, kk_bc, kk_W1, kk_W2,
           vs_Wc, vs_bc, vs_W1, vs_W2,
           f2_Wc, f2_bc, f2_W1, f2_W2,
           fd_w1, fd_b1, fd_w2, fd_b2,
           fg_w1, fg_b1, fg_w2, fg_b2):
    xyzT = jnp.swapaxes(xyz, 1, 2)

    WcT = jnp.stack([qs_Wc.T, kk_Wc.T, vs_Wc.T])
    bcs = jnp.stack([qs_bc, kk_bc, vs_bc])[:, None, :]
    W1s = jnp.stack([qs_W1, kk_W1, vs_W1])
    W2s = jnp.stack([qs_W2, kk_W2, vs_W2])

    x, q, p, table = pl.pallas_call(
        _k1_body,
        grid=(B,),
        in_specs=[
            pl.BlockSpec((1, N, D_PTS), lambda b: (b, 0, 0)),
            pl.BlockSpec((1, N, 3), lambda b: (b, 0, 0)),
            pl.BlockSpec((D_PTS, D), lambda b: (0, 0)),
            pl.BlockSpec((1, D), lambda b: (0, 0)),
            pl.BlockSpec((3, D, D), lambda b: (0, 0, 0)),
            pl.BlockSpec((3, 1, D), lambda b: (0, 0, 0)),
            pl.BlockSpec((3, D, D), lambda b: (0, 0, 0)),
            pl.BlockSpec((3, D, D), lambda b: (0, 0, 0)),
            pl.BlockSpec((3, D), lambda b: (0, 0)),
        ],
        out_specs=[
            pl.BlockSpec((1, N, D), lambda b: (b, 0, 0)),
            pl.BlockSpec((1, N, D), lambda b: (b, 0, 0)),
            pl.BlockSpec((1, N, D), lambda b: (b, 0, 0)),
            pl.BlockSpec((1, N, TW), lambda b: (b, 0, 0)),
        ],
        out_shape=[
            jax.ShapeDtypeStruct((B, N, D), F32),
            jax.ShapeDtypeStruct((B, N, D), F32),
            jax.ShapeDtypeStruct((B, N, D), F32),
            jax.ShapeDtypeStruct((B, N, TW), jnp.int32),
        ],
    )(features, xyz, fc1_w.T, fc1_b[None, :], WcT, bcs, W1s, W2s,
      fd_w1.T)

    knn_idx = pl.pallas_call(
        _k2_body,
        grid=(B, N // NB_TOPK),
        in_specs=[
            pl.BlockSpec((1, NB_TOPK, 3), lambda b, i: (b, i, 0)),
            pl.BlockSpec((1, 3, N), lambda b, i: (b, 0, 0)),
        ],
        out_specs=pl.BlockSpec((1, NB_TOPK, K), lambda b, i: (b, i, 0)),
        out_shape=jax.ShapeDtypeStruct((B, N, K), jnp.int32),
    )(xyz, xyzT)

    r_all = B * N * K
    g = _sc_gather(table.reshape(B * N, TW), knn_idx.reshape(r_all), r_all)
    g = g.reshape(B, N * K, TW)

    attn, res_pre = pl.pallas_call(
        _k4_body,
        grid=(B, N // NB_ATT),
        in_specs=[
            pl.BlockSpec((1, NB_ATT, D), lambda b, i: (b, i, 0)),
            pl.BlockSpec((1, NB_ATT * K, TW), lambda b, i: (b, i, 0)),
            pl.BlockSpec((1, NB_ATT, D), lambda b, i: (b, i, 0)),
            pl.BlockSpec((1, D), lambda b, i: (0, 0)),
            pl.BlockSpec((D, D), lambda b, i: (0, 0)),
            pl.BlockSpec((1, D), lambda b, i: (0, 0)),
            pl.BlockSpec((D, D), lambda b, i: (0, 0)),
            pl.BlockSpec((1, D), lambda b, i: (0, 0)),
            pl.BlockSpec((D, D), lambda b, i: (0, 0)),
            pl.BlockSpec((1, D), lambda b, i: (0, 0)),
        ],
        out_specs=[
            pl.BlockSpec((1, NB_ATT * K, D), lambda b, i: (b, i, 0)),
            pl.BlockSpec((1, NB_ATT, D), lambda b, i: (b, i, 0)),
        ],
        out_shape=[
            jax.ShapeDtypeStruct((B, N * K, D), F32),
            jax.ShapeDtypeStruct((B, N, D), F32),
        ],
    )(q, g, p,
      fd_b1[None, :], fd_w2.T, fd_b2[None, :],
      fg_w1.T, fg_b1[None, :], fg_w2.T, fg_b2[None, :])

    res = pl.pallas_call(
        _k5_body,
        grid=(B,),
        in_specs=[
            pl.BlockSpec((1, N, D), lambda b: (b, 0, 0)),
            pl.BlockSpec((1, N, D), lambda b: (b, 0, 0)),
            pl.BlockSpec((D, D), lambda b: (0, 0)),
            pl.BlockSpec((1, D), lambda b: (0, 0)),
            pl.BlockSpec((D, D), lambda b: (0, 0)),
            pl.BlockSpec((D, D), lambda b: (0, 0)),
        ],
        out_specs=pl.BlockSpec((1, N, D), lambda b: (b, 0, 0)),
        out_shape=jax.ShapeDtypeStruct((B, N, D), F32),
    )(res_pre, x, f2_Wc.T, f2_bc[None, :], f2_W1, f2_W2)

    return res, attn.reshape(B, N, K, D)


# speedup vs baseline: 1.1516x; 1.1516x over previous
import functools

import jax
import jax.numpy as jnp
from jax import lax
from jax.experimental import pallas as pl
from jax.experimental.pallas import tpu as pltpu
from jax.experimental.pallas import tpu_sc as plsc

B, N, D_PTS, D, K = 4, 2048, 64, 128, 24
F32 = jnp.float32


def _mlinear_body(feats, WcT, bc, W1, W2):
    mf = jnp.mean(feats, axis=0, keepdims=True)
    mf_rows = jnp.broadcast_to(mf, (D, D))
    mf_cols = mf_rows.T
    w = mf_rows * W1 - mf_cols * W2
    w = jnp.dot(w, WcT, preferred_element_type=F32) + bc
    denom = jnp.sum(jnp.abs(w) + 1e-05, axis=-1, keepdims=True)
    w = w / denom
    return jnp.dot(feats, w, preferred_element_type=F32)


def _k1_body(feats_ref, xyz_ref, fc1_wT_ref, fc1_b_ref,
             WcT_ref, bc_ref, W1_ref, W2_ref, fdw1T_ref,
             x_ref, q_ref, p_ref, tbl_ref):
    x = jnp.dot(feats_ref[0], fc1_wT_ref[...],
                preferred_element_type=F32) + fc1_b_ref[...]
    x_ref[0] = x
    q_ref[0] = _mlinear_body(x, WcT_ref[0], bc_ref[0], W1_ref[0], W2_ref[0])
    kf = _mlinear_body(x, WcT_ref[1], bc_ref[1], W1_ref[1], W2_ref[1])
    vf = _mlinear_body(x, WcT_ref[2], bc_ref[2], W1_ref[2], W2_ref[2])
    p = jnp.dot(xyz_ref[0], fdw1T_ref[...], preferred_element_type=F32)
    p_ref[0] = p
    def hi16(a):
        return jnp.bitwise_and(
            lax.bitcast_convert_type(
                a.astype(jnp.bfloat16).astype(F32), jnp.int32),
            jnp.int32(-65536))
    kv = jnp.bitwise_or(hi16(kf), lax.shift_right_logical(hi16(vf), 16))
    tbl_ref[0] = jnp.concatenate([kv, hi16(p)], axis=-1)


def _k5_body(res_ref, x_ref, WcT_ref, bc_ref, W1_ref, W2_ref, out_ref):
    out_ref[0] = _mlinear_body(res_ref[0], WcT_ref[...], bc_ref[...],
                               W1_ref[...], W2_ref[...]) + x_ref[0]


NB_TOPK = 512


def _k2_body(xyz_ref, xyzT_ref, idx_ref):
    b = pl.program_id(0)
    xyz_blk = xyz_ref[0]
    xyzT = xyzT_ref[0]
    dot = jnp.dot(xyz_blk, xyzT, preferred_element_type=F32)
    sq_blk = jnp.sum(xyz_blk * xyz_blk, axis=1, keepdims=True)
    sq_row = jnp.sum(xyzT * xyzT, axis=0, keepdims=True)
    d = (sq_blk + sq_row) - 2.0 * dot
    INF = jnp.float32(jnp.inf)
    NF = jnp.float32(N)
    iota = (lax.broadcasted_iota(jnp.int32, (NB_TOPK, N), 1)).astype(F32)
    cols = []
    for _ in range(K):
        m = jnp.min(d, axis=1, keepdims=True)
        j = jnp.min(jnp.where(d == m, iota, NF), axis=1, keepdims=True)
        cols.append(j)
        d = jnp.where(iota == j, INF, d)
    idx = jnp.concatenate(cols, axis=1).astype(jnp.int32)
    idx_ref[0] = idx + b * N


TW = 2 * D
NW = 32
CHUNK = 128


def _sc_gather(table, idx_flat, rows_total):
    rows_per_w = rows_total // NW
    n_chunks = rows_per_w // CHUNK
    mesh = plsc.VectorSubcoreMesh(core_axis_name="c", subcore_axis_name="s")

    @functools.partial(
        pl.kernel,
        mesh=mesh,
        out_type=jax.ShapeDtypeStruct((rows_total, TW), jnp.int32),
        scratch_types=[
            pltpu.VMEM((rows_per_w,), jnp.int32),
            pltpu.VMEM((2, CHUNK, TW), jnp.int32),
            pltpu.SemaphoreType.DMA,
            pltpu.SemaphoreType.DMA,
        ],
    )
    def gather_kernel(table_hbm, idx_hbm, out_hbm, idx_v, rows_v, s0, s1):
        wid = lax.axis_index("s") * 2 + lax.axis_index("c")
        w_base = wid * rows_per_w
        pltpu.sync_copy(idx_hbm.at[pl.ds(w_base, rows_per_w)], idx_v)
        sems = (s0, s1)

        def start(c, buf):
            pltpu.async_copy(
                table_hbm.at[idx_v.at[pl.ds(c * CHUNK, CHUNK)]],
                rows_v.at[buf], sems[buf])

        def finish(c, buf):
            pltpu.make_async_copy(
                table_hbm.at[idx_v.at[pl.ds(c * CHUNK, CHUNK)]],
                rows_v.at[buf], sems[buf]).wait()
            pltpu.sync_copy(rows_v.at[buf],
                            out_hbm.at[pl.ds(w_base + c * CHUNK, CHUNK)])

        start(0, 0)

        def body(i, carry):
            c = i * 2
            start(c + 1, 1)
            finish(c, 0)

            @pl.when(c + 2 < n_chunks)
            def _():
                start(c + 2, 0)

            finish(c + 1, 1)
            return carry

        lax.fori_loop(0, n_chunks // 2, body, 0)

    return gather_kernel(table, idx_flat)


NB_ATT = 128


def _k4_body(q_ref, g_ref, p_ref,
             fdb1_ref, fdw2T_ref, fdb2_ref,
             fgw1T_ref, fgb1_ref, fgw2T_ref, fgb2_ref,
             attn_ref, res_ref):
    rows = NB_ATT * K
    g = g_ref[0]
    kvg = g[:, :D]
    kg = lax.bitcast_convert_type(
        jnp.bitwise_and(kvg, jnp.int32(-65536)), F32)
    vg = lax.bitcast_convert_type(lax.shift_left(kvg, 16), F32)
    pg = lax.bitcast_convert_type(g[:, D:], F32)

    p_blk = p_ref[0]
    p_rep = jnp.reshape(
        jnp.broadcast_to(p_blk[:, None, :], (NB_ATT, K, D)), (rows, D))
    h1 = jax.nn.relu(p_rep - pg + fdb1_ref[...])
    pos = jnp.dot(h1, fdw2T_ref[...],
                  preferred_element_type=F32) + fdb2_ref[...]

    q_rep = jnp.reshape(
        jnp.broadcast_to(q_ref[0][:, None, :], (NB_ATT, K, D)), (rows, D))
    pre = q_rep - kg + pos
    h2 = jax.nn.relu(
        jnp.dot(pre, fgw1T_ref[...], preferred_element_type=F32)
        + fgb1_ref[...])
    attn = jnp.dot(h2, fgw2T_ref[...],
                   preferred_element_type=F32) + fgb2_ref[...]
    attn = attn / jnp.sum(jnp.abs(attn) + 1e-05, axis=-1, keepdims=True)
    attn_ref[0] = attn
    wv = attn * (vg + pos)
    res_ref[0] = jnp.sum(jnp.reshape(wv, (NB_ATT, K, D)), axis=1)


def kernel(features, xyz, fc1_w, fc1_b,
           qs_Wc, qs_bc, qs_W1, qs_W2,
           kk_Wc, kk_bc, kk_W1, kk_W2,
           vs_Wc, vs_bc, vs_W1, vs_W2,
           f2_Wc, f2_bc, f2_W1, f2_W2,
           fd_w1, fd_b1, fd_w2, fd_b2,
           fg_w1, fg_b1, fg_w2, fg_b2):
    xyzT = jnp.swapaxes(xyz, 1, 2)

    WcT = jnp.stack([qs_Wc.T, kk_Wc.T, vs_Wc.T])
    bcs = jnp.stack([qs_bc, kk_bc, vs_bc])[:, None, :]
    W1s = jnp.stack([qs_W1, kk_W1, vs_W1])
    W2s = jnp.stack([qs_W2, kk_W2, vs_W2])

    x, q, p, table = pl.pallas_call(
        _k1_body,
        grid=(B,),
        in_specs=[
            pl.BlockSpec((1, N, D_PTS), lambda b: (b, 0, 0)),
            pl.BlockSpec((1, N, 3), lambda b: (b, 0, 0)),
            pl.BlockSpec((D_PTS, D), lambda b: (0, 0)),
            pl.BlockSpec((1, D), lambda b: (0, 0)),
            pl.BlockSpec((3, D, D), lambda b: (0, 0, 0)),
            pl.BlockSpec((3, 1, D), lambda b: (0, 0, 0)),
            pl.BlockSpec((3, D, D), lambda b: (0, 0, 0)),
            pl.BlockSpec((3, D, D), lambda b: (0, 0, 0)),
            pl.BlockSpec((3, D), lambda b: (0, 0)),
        ],
        out_specs=[
            pl.BlockSpec((1, N, D), lambda b: (b, 0, 0)),
            pl.BlockSpec((1, N, D), lambda b: (b, 0, 0)),
            pl.BlockSpec((1, N, D), lambda b: (b, 0, 0)),
            pl.BlockSpec((1, N, TW), lambda b: (b, 0, 0)),
        ],
        out_shape=[
            jax.ShapeDtypeStruct((B, N, D), F32),
            jax.ShapeDtypeStruct((B, N, D), F32),
            jax.ShapeDtypeStruct((B, N, D), F32),
            jax.ShapeDtypeStruct((B, N, TW), jnp.int32),
        ],
    )(features, xyz, fc1_w.T, fc1_b[None, :], WcT, bcs, W1s, W2s,
      fd_w1.T)

    knn_idx = pl.pallas_call(
        _k2_body,
        grid=(B, N // NB_TOPK),
        in_specs=[
            pl.BlockSpec((1, NB_TOPK, 3), lambda b, i: (b, i, 0)),
            pl.BlockSpec((1, 3, N), lambda b, i: (b, 0, 0)),
        ],
        out_specs=pl.BlockSpec((1, NB_TOPK, K), lambda b, i: (b, i, 0)),
        out_shape=jax.ShapeDtypeStruct((B, N, K), jnp.int32),
    )(xyz, xyzT)

    r_all = B * N * K
    g = _sc_gather(table.reshape(B * N, TW), knn_idx.reshape(r_all), r_all)
    g = g.reshape(B, N * K, TW)

    attn, res_pre = pl.pallas_call(
        _k4_body,
        grid=(B, N // NB_ATT),
        in_specs=[
            pl.BlockSpec((1, NB_ATT, D), lambda b, i: (b, i, 0)),
            pl.BlockSpec((1, NB_ATT * K, TW), lambda b, i: (b, i, 0)),
            pl.BlockSpec((1, NB_ATT, D), lambda b, i: (b, i, 0)),
            pl.BlockSpec((1, D), lambda b, i: (0, 0)),
            pl.BlockSpec((D, D), lambda b, i: (0, 0)),
            pl.BlockSpec((1, D), lambda b, i: (0, 0)),
            pl.BlockSpec((D, D), lambda b, i: (0, 0)),
            pl.BlockSpec((1, D), lambda b, i: (0, 0)),
            pl.BlockSpec((D, D), lambda b, i: (0, 0)),
            pl.BlockSpec((1, D), lambda b, i: (0, 0)),
        ],
        out_specs=[
            pl.BlockSpec((1, NB_ATT * K, D), lambda b, i: (b, i, 0)),
            pl.BlockSpec((1, NB_ATT, D), lambda b, i: (b, i, 0)),
        ],
        out_shape=[
            jax.ShapeDtypeStruct((B, N * K, D), F32),
            jax.ShapeDtypeStruct((B, N, D), F32),
        ],
    )(q, g, p,
      fd_b1[None, :], fd_w2.T, fd_b2[None, :],
      fg_w1.T, fg_b1[None, :], fg_w2.T, fg_b2[None, :])

    res = pl.pallas_call(
        _k5_body,
        grid=(B,),
        in_specs=[
            pl.BlockSpec((1, N, D), lambda b: (b, 0, 0)),
            pl.BlockSpec((1, N, D), lambda b: (b, 0, 0)),
            pl.BlockSpec((D, D), lambda b: (0, 0)),
            pl.BlockSpec((1, D), lambda b: (0, 0)),
            pl.BlockSpec((D, D), lambda b: (0, 0)),
            pl.BlockSpec((D, D), lambda b: (0, 0)),
        ],
        out_specs=pl.BlockSpec((1, N, D), lambda b: (b, 0, 0)),
        out_shape=jax.ShapeDtypeStruct((B, N, D), F32),
    )(res_pre, x, f2_Wc.T, f2_bc[None, :], f2_W1, f2_W2)

    return res, attn.reshape(B, N, K, D)


# speedup vs baseline: 1.1843x; 1.0284x over previous
import functools

import jax
import jax.numpy as jnp
from jax import lax
from jax.experimental import pallas as pl
from jax.experimental.pallas import tpu as pltpu
from jax.experimental.pallas import tpu_sc as plsc

B, N, D_PTS, D, K = 4, 2048, 64, 128, 24
F32 = jnp.float32


def _mlinear_body(feats, WcT, bc, W1, W2):
    mf = jnp.mean(feats, axis=0, keepdims=True)
    mf_rows = jnp.broadcast_to(mf, (D, D))
    mf_cols = mf_rows.T
    w = mf_rows * W1 - mf_cols * W2
    w = jnp.dot(w, WcT, preferred_element_type=F32) + bc
    denom = jnp.sum(jnp.abs(w) + 1e-05, axis=-1, keepdims=True)
    w = w / denom
    return jnp.dot(feats, w, preferred_element_type=F32)


def _k1_body(feats_ref, xyz_ref, fc1_wT_ref, fc1_b_ref,
             WcT_ref, bc_ref, W1_ref, W2_ref, fdw1T_ref,
             x_ref, q_ref, p_ref, tbl_ref):
    x = jnp.dot(feats_ref[0], fc1_wT_ref[...],
                preferred_element_type=F32) + fc1_b_ref[...]
    x_ref[0] = x
    q_ref[0] = _mlinear_body(x, WcT_ref[0], bc_ref[0], W1_ref[0], W2_ref[0])
    kf = _mlinear_body(x, WcT_ref[1], bc_ref[1], W1_ref[1], W2_ref[1])
    vf = _mlinear_body(x, WcT_ref[2], bc_ref[2], W1_ref[2], W2_ref[2])
    p = jnp.dot(xyz_ref[0], fdw1T_ref[...], preferred_element_type=F32)
    p_ref[0] = p
    def hi16(a):
        return jnp.bitwise_and(
            lax.bitcast_convert_type(
                a.astype(jnp.bfloat16).astype(F32), jnp.int32),
            jnp.int32(-65536))
    kv = jnp.bitwise_or(hi16(kf), lax.shift_right_logical(hi16(vf), 16))
    tbl_ref[0] = jnp.concatenate([kv, hi16(p)], axis=-1)


def _k5_body(res_ref, x_ref, WcT_ref, bc_ref, W1_ref, W2_ref, out_ref):
    out_ref[0] = _mlinear_body(res_ref[0], WcT_ref[...], bc_ref[...],
                               W1_ref[...], W2_ref[...]) + x_ref[0]


NB_TOPK = 512


def _k2_body(xyz_ref, xyzT_ref, idx_ref):
    b = pl.program_id(0)
    xyz_blk = xyz_ref[0]
    xyzT = xyzT_ref[0]
    dot = jnp.dot(xyz_blk, xyzT, preferred_element_type=F32)
    sq_blk = jnp.sum(xyz_blk * xyz_blk, axis=1, keepdims=True)
    sq_row = jnp.sum(xyzT * xyzT, axis=0, keepdims=True)
    d = (sq_blk + sq_row) - 2.0 * dot
    INF = jnp.float32(jnp.inf)
    NF = jnp.float32(N)
    iota = (lax.broadcasted_iota(jnp.int32, (NB_TOPK, N), 1)).astype(F32)
    cols = []
    for _ in range(K):
        m = jnp.min(d, axis=1, keepdims=True)
        j = jnp.min(jnp.where(d == m, iota, NF), axis=1, keepdims=True)
        cols.append(j)
        d = jnp.where(iota == j, INF, d)
    idx = jnp.concatenate(cols, axis=1).astype(jnp.int32)
    idx_ref[0] = idx + b * N


TW = 2 * D
NW = 32
CHUNK = 128


def _sc_gather(table, idx_flat, rows_total):
    rows_per_w = rows_total // NW
    n_chunks = rows_per_w // CHUNK
    mesh = plsc.VectorSubcoreMesh(core_axis_name="c", subcore_axis_name="s")

    @functools.partial(
        pl.kernel,
        mesh=mesh,
        out_type=jax.ShapeDtypeStruct((rows_total, TW), jnp.int32),
        scratch_types=[
            pltpu.VMEM((rows_per_w,), jnp.int32),
            pltpu.VMEM((2, CHUNK, TW), jnp.int32),
            pltpu.SemaphoreType.DMA,
            pltpu.SemaphoreType.DMA,
        ],
    )
    def gather_kernel(table_hbm, idx_hbm, out_hbm, idx_v, rows_v, s0, s1):
        wid = lax.axis_index("s") * 2 + lax.axis_index("c")
        w_base = wid * rows_per_w
        pltpu.sync_copy(idx_hbm.at[pl.ds(w_base, rows_per_w)], idx_v)
        sems = (s0, s1)

        def start(c, buf):
            pltpu.async_copy(
                table_hbm.at[idx_v.at[pl.ds(c * CHUNK, CHUNK)]],
                rows_v.at[buf], sems[buf])

        def finish(c, buf):
            pltpu.make_async_copy(
                table_hbm.at[idx_v.at[pl.ds(c * CHUNK, CHUNK)]],
                rows_v.at[buf], sems[buf]).wait()
            pltpu.sync_copy(rows_v.at[buf],
                            out_hbm.at[pl.ds(w_base + c * CHUNK, CHUNK)])

        start(0, 0)

        def body(i, carry):
            c = i * 2
            start(c + 1, 1)
            finish(c, 0)

            @pl.when(c + 2 < n_chunks)
            def _():
                start(c + 2, 0)

            finish(c + 1, 1)
            return carry

        lax.fori_loop(0, n_chunks // 2, body, 0)

    return gather_kernel(table, idx_flat)


NB_ATT = 256


def _k4_body(q_ref, g_ref, p_ref,
             fdb1_ref, fdw2T_ref, fdb2_ref,
             fgw1T_ref, fgb1_ref, fgw2T_ref, fgb2_ref,
             attn_ref, res_ref):
    rows = NB_ATT * K
    g = g_ref[0]
    kvg = g[:, :D]
    kg = lax.bitcast_convert_type(
        jnp.bitwise_and(kvg, jnp.int32(-65536)), F32)
    vg = lax.bitcast_convert_type(lax.shift_left(kvg, 16), F32)
    pg = lax.bitcast_convert_type(g[:, D:], F32)

    p_blk = p_ref[0]
    p_rep = jnp.reshape(
        jnp.broadcast_to(p_blk[:, None, :], (NB_ATT, K, D)), (rows, D))
    h1 = jax.nn.relu(p_rep - pg + fdb1_ref[...])
    pos = jnp.dot(h1, fdw2T_ref[...],
                  preferred_element_type=F32) + fdb2_ref[...]

    q_rep = jnp.reshape(
        jnp.broadcast_to(q_ref[0][:, None, :], (NB_ATT, K, D)), (rows, D))
    pre = q_rep - kg + pos
    h2 = jax.nn.relu(
        jnp.dot(pre, fgw1T_ref[...], preferred_element_type=F32)
        + fgb1_ref[...])
    attn = jnp.dot(h2, fgw2T_ref[...],
                   preferred_element_type=F32) + fgb2_ref[...]
    attn = attn / jnp.sum(jnp.abs(attn) + 1e-05, axis=-1, keepdims=True)
    attn_ref[0] = attn
    wv = attn * (vg + pos)
    res_ref[0] = jnp.sum(jnp.reshape(wv, (NB_ATT, K, D)), axis=1)


def kernel(features, xyz, fc1_w, fc1_b,
           qs_Wc, qs_bc, qs_W1, qs_W2,
           kk_Wc, kk_bc, kk_W1, kk_W2,
           vs_Wc, vs_bc, vs_W1, vs_W2,
           f2_Wc, f2_bc, f2_W1, f2_W2,
           fd_w1, fd_b1, fd_w2, fd_b2,
           fg_w1, fg_b1, fg_w2, fg_b2):
    xyzT = jnp.swapaxes(xyz, 1, 2)

    WcT = jnp.stack([qs_Wc.T, kk_Wc.T, vs_Wc.T])
    bcs = jnp.stack([qs_bc, kk_bc, vs_bc])[:, None, :]
    W1s = jnp.stack([qs_W1, kk_W1, vs_W1])
    W2s = jnp.stack([qs_W2, kk_W2, vs_W2])

    x, q, p, table = pl.pallas_call(
        _k1_body,
        grid=(B,),
        in_specs=[
            pl.BlockSpec((1, N, D_PTS), lambda b: (b, 0, 0)),
            pl.BlockSpec((1, N, 3), lambda b: (b, 0, 0)),
            pl.BlockSpec((D_PTS, D), lambda b: (0, 0)),
            pl.BlockSpec((1, D), lambda b: (0, 0)),
            pl.BlockSpec((3, D, D), lambda b: (0, 0, 0)),
            pl.BlockSpec((3, 1, D), lambda b: (0, 0, 0)),
            pl.BlockSpec((3, D, D), lambda b: (0, 0, 0)),
            pl.BlockSpec((3, D, D), lambda b: (0, 0, 0)),
            pl.BlockSpec((3, D), lambda b: (0, 0)),
        ],
        out_specs=[
            pl.BlockSpec((1, N, D), lambda b: (b, 0, 0)),
            pl.BlockSpec((1, N, D), lambda b: (b, 0, 0)),
            pl.BlockSpec((1, N, D), lambda b: (b, 0, 0)),
            pl.BlockSpec((1, N, TW), lambda b: (b, 0, 0)),
        ],
        out_shape=[
            jax.ShapeDtypeStruct((B, N, D), F32),
            jax.ShapeDtypeStruct((B, N, D), F32),
            jax.ShapeDtypeStruct((B, N, D), F32),
            jax.ShapeDtypeStruct((B, N, TW), jnp.int32),
        ],
    )(features, xyz, fc1_w.T, fc1_b[None, :], WcT, bcs, W1s, W2s,
      fd_w1.T)

    knn_idx = pl.pallas_call(
        _k2_body,
        grid=(B, N // NB_TOPK),
        in_specs=[
            pl.BlockSpec((1, NB_TOPK, 3), lambda b, i: (b, i, 0)),
            pl.BlockSpec((1, 3, N), lambda b, i: (b, 0, 0)),
        ],
        out_specs=pl.BlockSpec((1, NB_TOPK, K), lambda b, i: (b, i, 0)),
        out_shape=jax.ShapeDtypeStruct((B, N, K), jnp.int32),
    )(xyz, xyzT)

    r_all = B * N * K
    g = _sc_gather(table.reshape(B * N, TW), knn_idx.reshape(r_all), r_all)
    g = g.reshape(B, N * K, TW)

    attn, res_pre = pl.pallas_call(
        _k4_body,
        grid=(B, N // NB_ATT),
        in_specs=[
            pl.BlockSpec((1, NB_ATT, D), lambda b, i: (b, i, 0)),
            pl.BlockSpec((1, NB_ATT * K, TW), lambda b, i: (b, i, 0)),
            pl.BlockSpec((1, NB_ATT, D), lambda b, i: (b, i, 0)),
            pl.BlockSpec((1, D), lambda b, i: (0, 0)),
            pl.BlockSpec((D, D), lambda b, i: (0, 0)),
            pl.BlockSpec((1, D), lambda b, i: (0, 0)),
            pl.BlockSpec((D, D), lambda b, i: (0, 0)),
            pl.BlockSpec((1, D), lambda b, i: (0, 0)),
            pl.BlockSpec((D, D), lambda b, i: (0, 0)),
            pl.BlockSpec((1, D), lambda b, i: (0, 0)),
        ],
        out_specs=[
            pl.BlockSpec((1, NB_ATT * K, D), lambda b, i: (b, i, 0)),
            pl.BlockSpec((1, NB_ATT, D), lambda b, i: (b, i, 0)),
        ],
        out_shape=[
            jax.ShapeDtypeStruct((B, N * K, D), F32),
            jax.ShapeDtypeStruct((B, N, D), F32),
        ],
    )(q, g, p,
      fd_b1[None, :], fd_w2.T, fd_b2[None, :],
      fg_w1.T, fg_b1[None, :], fg_w2.T, fg_b2[None, :])

    res = pl.pallas_call(
        _k5_body,
        grid=(B,),
        in_specs=[
            pl.BlockSpec((1, N, D), lambda b: (b, 0, 0)),
            pl.BlockSpec((1, N, D), lambda b: (b, 0, 0)),
            pl.BlockSpec((D, D), lambda b: (0, 0)),
            pl.BlockSpec((1, D), lambda b: (0, 0)),
            pl.BlockSpec((D, D), lambda b: (0, 0)),
            pl.BlockSpec((D, D), lambda b: (0, 0)),
        ],
        out_specs=pl.BlockSpec((1, N, D), lambda b: (b, 0, 0)),
        out_shape=jax.ShapeDtypeStruct((B, N, D), F32),
    )(res_pre, x, f2_Wc.T, f2_bc[None, :], f2_W1, f2_W2)

    return res, attn.reshape(B, N, K, D)


# speedup vs baseline: 1.1881x; 1.0032x over previous
import functools

import jax
import jax.numpy as jnp
from jax import lax
from jax.experimental import pallas as pl
from jax.experimental.pallas import tpu as pltpu
from jax.experimental.pallas import tpu_sc as plsc

B, N, D_PTS, D, K = 4, 2048, 64, 128, 24
F32 = jnp.float32


def _mlinear_body(feats, WcT, bc, W1, W2):
    mf = jnp.mean(feats, axis=0, keepdims=True)
    mf_rows = jnp.broadcast_to(mf, (D, D))
    mf_cols = mf_rows.T
    w = mf_rows * W1 - mf_cols * W2
    w = jnp.dot(w, WcT, preferred_element_type=F32) + bc
    denom = jnp.sum(jnp.abs(w) + 1e-05, axis=-1, keepdims=True)
    w = w / denom
    return jnp.dot(feats, w, preferred_element_type=F32)


def _k1_body(feats_ref, xyz_ref, fc1_wT_ref, fc1_b_ref,
             WcT_ref, bc_ref, W1_ref, W2_ref, fdw1T_ref,
             x_ref, q_ref, p_ref, tbl_ref):
    x = jnp.dot(feats_ref[0], fc1_wT_ref[...],
                preferred_element_type=F32) + fc1_b_ref[...]
    x_ref[0] = x
    q_ref[0] = _mlinear_body(x, WcT_ref[0], bc_ref[0], W1_ref[0], W2_ref[0])
    kf = _mlinear_body(x, WcT_ref[1], bc_ref[1], W1_ref[1], W2_ref[1])
    vf = _mlinear_body(x, WcT_ref[2], bc_ref[2], W1_ref[2], W2_ref[2])
    p = jnp.dot(xyz_ref[0], fdw1T_ref[...], preferred_element_type=F32)
    p_ref[0] = p
    def hi16(a):
        return jnp.bitwise_and(
            lax.bitcast_convert_type(
                a.astype(jnp.bfloat16).astype(F32), jnp.int32),
            jnp.int32(-65536))
    kv = jnp.bitwise_or(hi16(kf), lax.shift_right_logical(hi16(vf), 16))
    tbl_ref[0] = jnp.concatenate([kv, hi16(p)], axis=-1)


def _k5_body(res_ref, x_ref, WcT_ref, bc_ref, W1_ref, W2_ref, out_ref):
    out_ref[0] = _mlinear_body(res_ref[0], WcT_ref[...], bc_ref[...],
                               W1_ref[...], W2_ref[...]) + x_ref[0]


NB_TOPK = 512


def _k2_body(xyz_ref, xyzT_ref, idx_ref):
    b = pl.program_id(0)
    xyz_blk = xyz_ref[0]
    xyzT = xyzT_ref[0]
    dot = jnp.dot(xyz_blk, xyzT, preferred_element_type=F32)
    sq_blk = jnp.sum(xyz_blk * xyz_blk, axis=1, keepdims=True)
    sq_row = jnp.sum(xyzT * xyzT, axis=0, keepdims=True)
    d = (sq_blk + sq_row) - 2.0 * dot
    INF = jnp.float32(jnp.inf)
    NF = jnp.float32(N)
    iota = (lax.broadcasted_iota(jnp.int32, (NB_TOPK, N), 1)).astype(F32)
    cols = []
    for _ in range(K):
        m = jnp.min(d, axis=1, keepdims=True)
        j = jnp.min(jnp.where(d == m, iota, NF), axis=1, keepdims=True)
        cols.append(j)
        d = jnp.where(iota == j, INF, d)
    idx = jnp.concatenate(cols, axis=1).astype(jnp.int32)
    idx_ref[0] = idx + b * N


TW = 2 * D
NW = 32
CHUNK = 128


def _sc_gather(table, idx_flat, rows_total):
    rows_per_w = rows_total // NW
    n_chunks = rows_per_w // CHUNK
    mesh = plsc.VectorSubcoreMesh(core_axis_name="c", subcore_axis_name="s")

    @functools.partial(
        pl.kernel,
        mesh=mesh,
        out_type=jax.ShapeDtypeStruct((rows_total, TW), jnp.int32),
        scratch_types=[
            pltpu.VMEM((rows_per_w,), jnp.int32),
            pltpu.VMEM((2, CHUNK, TW), jnp.int32),
            pltpu.SemaphoreType.DMA,
            pltpu.SemaphoreType.DMA,
        ],
    )
    def gather_kernel(table_hbm, idx_hbm, out_hbm, idx_v, rows_v, s0, s1):
        wid = lax.axis_index("s") * 2 + lax.axis_index("c")
        w_base = wid * rows_per_w
        pltpu.sync_copy(idx_hbm.at[pl.ds(w_base, rows_per_w)], idx_v)
        sems = (s0, s1)

        def start(c, buf):
            pltpu.async_copy(
                table_hbm.at[idx_v.at[pl.ds(c * CHUNK, CHUNK)]],
                rows_v.at[buf], sems[buf])

        def finish(c, buf):
            pltpu.make_async_copy(
                table_hbm.at[idx_v.at[pl.ds(c * CHUNK, CHUNK)]],
                rows_v.at[buf], sems[buf]).wait()
            pltpu.sync_copy(rows_v.at[buf],
                            out_hbm.at[pl.ds(w_base + c * CHUNK, CHUNK)])

        start(0, 0)

        def body(i, carry):
            c = i * 2
            start(c + 1, 1)
            finish(c, 0)

            @pl.when(c + 2 < n_chunks)
            def _():
                start(c + 2, 0)

            finish(c + 1, 1)
            return carry

        lax.fori_loop(0, n_chunks // 2, body, 0)

    return gather_kernel(table, idx_flat)


NB_ATT = 512


def _k4_body(q_ref, g_ref, p_ref,
             fdb1_ref, fdw2T_ref, fdb2_ref,
             fgw1T_ref, fgb1_ref, fgw2T_ref, fgb2_ref,
             attn_ref, res_ref):
    rows = NB_ATT * K
    g = g_ref[0]
    kvg = g[:, :D]
    kg = lax.bitcast_convert_type(
        jnp.bitwise_and(kvg, jnp.int32(-65536)), F32)
    vg = lax.bitcast_convert_type(lax.shift_left(kvg, 16), F32)
    pg = lax.bitcast_convert_type(g[:, D:], F32)

    p_blk = p_ref[0]
    p_rep = jnp.reshape(
        jnp.broadcast_to(p_blk[:, None, :], (NB_ATT, K, D)), (rows, D))
    h1 = jax.nn.relu(p_rep - pg + fdb1_ref[...])
    pos = jnp.dot(h1, fdw2T_ref[...],
                  preferred_element_type=F32) + fdb2_ref[...]

    q_rep = jnp.reshape(
        jnp.broadcast_to(q_ref[0][:, None, :], (NB_ATT, K, D)), (rows, D))
    pre = q_rep - kg + pos
    h2 = jax.nn.relu(
        jnp.dot(pre, fgw1T_ref[...], preferred_element_type=F32)
        + fgb1_ref[...])
    attn = jnp.dot(h2, fgw2T_ref[...],
                   preferred_element_type=F32) + fgb2_ref[...]
    attn = attn / jnp.sum(jnp.abs(attn) + 1e-05, axis=-1, keepdims=True)
    attn_ref[0] = attn
    wv = attn * (vg + pos)
    res_ref[0] = jnp.sum(jnp.reshape(wv, (NB_ATT, K, D)), axis=1)


def kernel(features, xyz, fc1_w, fc1_b,
           qs_Wc, qs_bc, qs_W1, qs_W2,
           kk_Wc, kk_bc, kk_W1, kk_W2,
           vs_Wc, vs_bc, vs_W1, vs_W2,
           f2_Wc, f2_bc, f2_W1, f2_W2,
           fd_w1, fd_b1, fd_w2, fd_b2,
           fg_w1, fg_b1, fg_w2, fg_b2):
    xyzT = jnp.swapaxes(xyz, 1, 2)

    WcT = jnp.stack([qs_Wc.T, kk_Wc.T, vs_Wc.T])
    bcs = jnp.stack([qs_bc, kk_bc, vs_bc])[:, None, :]
    W1s = jnp.stack([qs_W1, kk_W1, vs_W1])
    W2s = jnp.stack([qs_W2, kk_W2, vs_W2])

    x, q, p, table = pl.pallas_call(
        _k1_body,
        grid=(B,),
        in_specs=[
            pl.BlockSpec((1, N, D_PTS), lambda b: (b, 0, 0)),
            pl.BlockSpec((1, N, 3), lambda b: (b, 0, 0)),
            pl.BlockSpec((D_PTS, D), lambda b: (0, 0)),
            pl.BlockSpec((1, D), lambda b: (0, 0)),
            pl.BlockSpec((3, D, D), lambda b: (0, 0, 0)),
            pl.BlockSpec((3, 1, D), lambda b: (0, 0, 0)),
            pl.BlockSpec((3, D, D), lambda b: (0, 0, 0)),
            pl.BlockSpec((3, D, D), lambda b: (0, 0, 0)),
            pl.BlockSpec((3, D), lambda b: (0, 0)),
        ],
        out_specs=[
            pl.BlockSpec((1, N, D), lambda b: (b, 0, 0)),
            pl.BlockSpec((1, N, D), lambda b: (b, 0, 0)),
            pl.BlockSpec((1, N, D), lambda b: (b, 0, 0)),
            pl.BlockSpec((1, N, TW), lambda b: (b, 0, 0)),
        ],
        out_shape=[
            jax.ShapeDtypeStruct((B, N, D), F32),
            jax.ShapeDtypeStruct((B, N, D), F32),
            jax.ShapeDtypeStruct((B, N, D), F32),
            jax.ShapeDtypeStruct((B, N, TW), jnp.int32),
        ],
    )(features, xyz, fc1_w.T, fc1_b[None, :], WcT, bcs, W1s, W2s,
      fd_w1.T)

    knn_idx = pl.pallas_call(
        _k2_body,
        grid=(B, N // NB_TOPK),
        in_specs=[
            pl.BlockSpec((1, NB_TOPK, 3), lambda b, i: (b, i, 0)),
            pl.BlockSpec((1, 3, N), lambda b, i: (b, 0, 0)),
        ],
        out_specs=pl.BlockSpec((1, NB_TOPK, K), lambda b, i: (b, i, 0)),
        out_shape=jax.ShapeDtypeStruct((B, N, K), jnp.int32),
    )(xyz, xyzT)

    r_all = B * N * K
    g = _sc_gather(table.reshape(B * N, TW), knn_idx.reshape(r_all), r_all)
    g = g.reshape(B, N * K, TW)

    attn, res_pre = pl.pallas_call(
        _k4_body,
        grid=(B, N // NB_ATT),
        in_specs=[
            pl.BlockSpec((1, NB_ATT, D), lambda b, i: (b, i, 0)),
            pl.BlockSpec((1, NB_ATT * K, TW), lambda b, i: (b, i, 0)),
            pl.BlockSpec((1, NB_ATT, D), lambda b, i: (b, i, 0)),
            pl.BlockSpec((1, D), lambda b, i: (0, 0)),
            pl.BlockSpec((D, D), lambda b, i: (0, 0)),
            pl.BlockSpec((1, D), lambda b, i: (0, 0)),
            pl.BlockSpec((D, D), lambda b, i: (0, 0)),
            pl.BlockSpec((1, D), lambda b, i: (0, 0)),
            pl.BlockSpec((D, D), lambda b, i: (0, 0)),
            pl.BlockSpec((1, D), lambda b, i: (0, 0)),
        ],
        out_specs=[
            pl.BlockSpec((1, NB_ATT * K, D), lambda b, i: (b, i, 0)),
            pl.BlockSpec((1, NB_ATT, D), lambda b, i: (b, i, 0)),
        ],
        out_shape=[
            jax.ShapeDtypeStruct((B, N * K, D), F32),
            jax.ShapeDtypeStruct((B, N, D), F32),
        ],
    )(q, g, p,
      fd_b1[None, :], fd_w2.T, fd_b2[None, :],
      fg_w1.T, fg_b1[None, :], fg_w2.T, fg_b2[None, :])

    res = pl.pallas_call(
        _k5_body,
        grid=(B,),
        in_specs=[
            pl.BlockSpec((1, N, D), lambda b: (b, 0, 0)),
            pl.BlockSpec((1, N, D), lambda b: (b, 0, 0)),
            pl.BlockSpec((D, D), lambda b: (0, 0)),
            pl.BlockSpec((1, D), lambda b: (0, 0)),
            pl.BlockSpec((D, D), lambda b: (0, 0)),
            pl.BlockSpec((D, D), lambda b: (0, 0)),
        ],
        out_specs=pl.BlockSpec((1, N, D), lambda b: (b, 0, 0)),
        out_shape=jax.ShapeDtypeStruct((B, N, D), F32),
    )(res_pre, x, f2_Wc.T, f2_bc[None, :], f2_W1, f2_W2)

    return res, attn.reshape(B, N, K, D)
